# baseline (speedup 1.0000x reference)
import functools

import jax
import jax.numpy as jnp
from jax import lax
from jax.experimental import pallas as pl
from jax.experimental.pallas import tpu as pltpu
from jax.experimental.pallas import tpu_sc as plsc

N_USER = 100000
N_ITEM = 1000000
D = 32
B = 4096
NN = 256
K = 10
NEG_WEIGHT = 200.0
GAMMA = 1e-4
LAMBDA = 2.75

NC = 2
NS = 16
NW = NC * NS
BW = B // NW
NEG_TOTAL = B * NN
GW = 128

_sc_mesh = plsc.VectorSubcoreMesh(core_axis_name="c", subcore_axis_name="s")


@functools.partial(
    pl.kernel,
    mesh=_sc_mesh,
    compiler_params=pltpu.CompilerParams(use_tc_tiling_on_sc=False),
    out_type=[
        jax.ShapeDtypeStruct((B, D), jnp.float32),
        jax.ShapeDtypeStruct((K + 1, B, D), jnp.float32),
        jax.ShapeDtypeStruct((K, B), jnp.float32),
        jax.ShapeDtypeStruct((NEG_TOTAL, D), jnp.float32),
    ],
    scratch_types=[
        pltpu.VMEM((BW,), jnp.int32),
        pltpu.VMEM((BW,), jnp.int32),
        pltpu.VMEM((BW, D), jnp.float32),
        pltpu.VMEM((K, BW), jnp.int32),
        pltpu.VMEM((K + 1, BW), jnp.int32),
        pltpu.VMEM((K, BW), jnp.float32),
        pltpu.VMEM((K + 1, BW, D), jnp.float32),
        pltpu.SemaphoreType.DMA,
    ],
)
def _sc_gather(users_hbm, pos_hbm, negidx_hbm, utab_hbm, itab_hbm,
               neif_hbm, conf_hbm,
               ue_out, pi_out, sim_out, neg_out,
               uidx_v, pidx_v, urows_v, neiidx_v, piidx_v, simv_v, pirows_v,
               sem):
    wid = lax.axis_index("s") * NC + lax.axis_index("c")
    base = wid * BW

    pltpu.sync_copy(users_hbm.at[pl.ds(base, BW)], uidx_v)
    pltpu.async_copy(utab_hbm.at[uidx_v], urows_v, sem).wait()
    pltpu.sync_copy(urows_v, ue_out.at[pl.ds(base, BW)])

    pltpu.sync_copy(pos_hbm.at[pl.ds(base, BW)], pidx_v)

    @pl.loop(0, BW, step=16)
    def _(i):
        p = pidx_v[pl.ds(i, 16)]
        piidx_v[0, pl.ds(i, 16)] = p
        for k in range(K):
            neiidx_v[k, pl.ds(i, 16)] = p * K + k

    cps = []
    for k in range(K):
        cps.append(pltpu.async_copy(neif_hbm.at[neiidx_v.at[k]],
                                    piidx_v.at[k + 1], sem))
        cps.append(pltpu.async_copy(conf_hbm.at[neiidx_v.at[k]],
                                    simv_v.at[k], sem))
    for c in cps:
        c.wait()
    for k in range(K):
        pltpu.sync_copy(simv_v.at[k], sim_out.at[k, pl.ds(base, BW)])

    cps = [pltpu.async_copy(itab_hbm.at[piidx_v.at[s]], pirows_v.at[s], sem)
           for s in range(K + 1)]
    for c in cps:
        c.wait()
    for s in range(K + 1):
        pltpu.sync_copy(pirows_v.at[s], pi_out.at[s, pl.ds(base, BW)])

    def body(i_vmem, o_vmem):
        pltpu.sync_copy(itab_hbm.at[i_vmem.at[0]], o_vmem)

    pltpu.emit_pipeline(
        body,
        grid=(NEG_TOTAL // GW,),
        in_specs=[pl.BlockSpec((1, GW), lambda i: (0, i))],
        out_specs=[pl.BlockSpec((GW, D), lambda i: (i, 0))],
        core_axis_name=("c", "s"),
        dimension_semantics=(pltpu.PARALLEL,),
    )(negidx_hbm, neg_out)


def _bce_pos(x):
    return jnp.maximum(x, 0.0) - x + jnp.log1p(jnp.exp(-jnp.abs(x)))


def _bce_neg(x):
    return jnp.maximum(x, 0.0) + jnp.log1p(jnp.exp(-jnp.abs(x)))


BBLK = 128


def _loss_body(ue_ref, pi_ref, sim_ref, neg_ref, out_ref):
    i = pl.program_id(0)
    ue = ue_ref[...]
    pi = pi_ref[...]
    sim = sim_ref[...]
    neg = neg_ref[...]

    pos_scores = jnp.sum(ue * pi[0], axis=-1)
    neg_scores = jnp.sum(ue[:, None, :] * neg, axis=-1)
    nbr_scores = jnp.sum(ue[None, :, :] * pi[1:], axis=-1)

    part = jnp.sum(_bce_pos(pos_scores))
    part += (NEG_WEIGHT / NN) * jnp.sum(_bce_neg(neg_scores))
    part += LAMBDA * jnp.sum(sim * _bce_neg(-nbr_scores))

    @pl.when(i == 0)
    def _():
        out_ref[0, 0] = 0.0

    out_ref[0, 0] += part


def _loss_call(ue, pi, sim, neg3):
    return pl.pallas_call(
        _loss_body,
        grid=(B // BBLK,),
        in_specs=[
            pl.BlockSpec((BBLK, D), lambda i: (i, 0)),
            pl.BlockSpec((K + 1, BBLK, D), lambda i: (0, i, 0)),
            pl.BlockSpec((K, BBLK), lambda i: (0, i)),
            pl.BlockSpec((BBLK, NN, D), lambda i: (i, 0, 0)),
        ],
        out_specs=pl.BlockSpec(memory_space=pltpu.SMEM),
        out_shape=jax.ShapeDtypeStruct((1, 1), jnp.float32),
    )(ue, pi, sim, neg3)


RB_I = 10000
RB_U = 1000


def _norm_body(itab_ref, utab_ref, out_ref):
    i = pl.program_id(0)
    part = jnp.sum(itab_ref[...] * itab_ref[...])
    part += jnp.sum(utab_ref[...] * utab_ref[...])

    @pl.when(i == 0)
    def _():
        out_ref[0, 0] = 0.0

    out_ref[0, 0] += part


def _norm_call(item_table, user_table):
    return pl.pallas_call(
        _norm_body,
        grid=(N_ITEM // RB_I,),
        in_specs=[
            pl.BlockSpec((RB_I, D), lambda i: (i, 0)),
            pl.BlockSpec((RB_U, D), lambda i: (i, 0)),
        ],
        out_specs=pl.BlockSpec(memory_space=pltpu.SMEM),
        out_shape=jax.ShapeDtypeStruct((1, 1), jnp.float32),
    )(item_table, user_table)


def kernel(users, pos_items, neg_items, user_table, item_table,
           ii_neighbor_mat, ii_constraint_mat):
    users = users.astype(jnp.int32)
    pos_items = pos_items.astype(jnp.int32)
    neg_flat = neg_items.astype(jnp.int32).reshape(1, NEG_TOTAL)
    nei_flat = ii_neighbor_mat.astype(jnp.int32).reshape(N_ITEM * K)
    con_flat = ii_constraint_mat.reshape(N_ITEM * K)

    ue, pi, sim, neg_rows = _sc_gather(
        users, pos_items, neg_flat, user_table, item_table, nei_flat,
        con_flat)
    neg3 = neg_rows.reshape(B, NN, D)

    loss_main = _loss_call(ue, pi, sim, neg3)[0, 0]
    ss = _norm_call(item_table, user_table)[0, 0]
    return loss_main + jnp.float32(GAMMA * 0.5) * ss
